# baseline (device time: 43453 ns/iter reference)
import jax
import jax.numpy as jnp
from jax import lax
from jax.experimental import pallas as pl
from jax.experimental.pallas import tpu as pltpu

N_DEV = 8


def kernel(x, router_W, route_idx, expert_W):
    n, d_model = x.shape
    e_total = router_W.shape[1]
    e_local, _, h_dim = expert_W.shape
    m_per = n // N_DEV

    def body(x_ref, rw_ref, idx_ref, ew_ref, out_ref,
             acc_ref, comm_ref, send_sems, recv_sems):
        my = lax.axis_index("i")
        left = lax.rem(my + N_DEV - 1, N_DEV)
        right = lax.rem(my + 1, N_DEV)

        barrier_sem = pltpu.get_barrier_semaphore()
        for nbr in (left, right):
            pl.semaphore_signal(
                barrier_sem, inc=1,
                device_id=(nbr,), device_id_type=pl.DeviceIdType.MESH,
            )
        pl.semaphore_wait(barrier_sem, 2)

        xv = x_ref[:, :]
        scores = jnp.dot(xv, rw_ref[:, :], preferred_element_type=jnp.float32)
        scores = scores - jnp.max(scores, axis=-1, keepdims=True)
        p = jnp.exp(scores)
        p = p / jnp.sum(p, axis=-1, keepdims=True)

        e0 = idx_ref[:, 0:1]
        e1 = idx_ref[:, 1:2]
        iota = lax.broadcasted_iota(jnp.int32, (n, e_total), 1)
        in_top2 = (iota == e0) | (iota == e1)
        g0 = jnp.sum(jnp.where(iota == e0, p, 0.0), axis=-1, keepdims=True)
        g1 = jnp.sum(jnp.where(iota == e1, p, 0.0), axis=-1, keepdims=True)
        gates = jnp.where(in_top2, p, 0.0) / (g0 + g1)

        acc = jnp.zeros((n, h_dim), jnp.float32)
        for le in range(e_local):
            ge = my * e_local + le
            gcol = jnp.sum(jnp.where(iota == ge, gates, 0.0),
                           axis=-1, keepdims=True)
            acc = acc + jnp.dot(gcol * xv, ew_ref[le],
                                preferred_element_type=jnp.float32)
        acc_ref[:, :] = acc

        c0 = lax.rem(my + N_DEV - 1, N_DEV)
        comm_ref[0, :, :] = acc_ref[pl.ds(c0 * m_per, m_per), :]
        for hop in range(N_DEV - 1):
            rdma = pltpu.make_async_remote_copy(
                src_ref=comm_ref.at[hop],
                dst_ref=comm_ref.at[hop + 1],
                send_sem=send_sems.at[hop],
                recv_sem=recv_sems.at[hop],
                device_id=(right,),
                device_id_type=pl.DeviceIdType.MESH,
            )
            rdma.start()
            rdma.wait()
            c = lax.rem(my + 2 * N_DEV - 2 - hop, N_DEV)
            blk = acc_ref[pl.ds(c * m_per, m_per), :]
            if hop < N_DEV - 2:
                comm_ref[hop + 1, :, :] = comm_ref[hop + 1, :, :] + blk
            else:
                out_ref[:, :] = comm_ref[hop + 1, :, :] + blk

    return pl.pallas_call(
        body,
        out_shape=jax.ShapeDtypeStruct((m_per, h_dim), jnp.float32),
        in_specs=[pl.BlockSpec(memory_space=pltpu.VMEM)] * 4,
        out_specs=pl.BlockSpec(memory_space=pltpu.VMEM),
        scratch_shapes=[
            pltpu.VMEM((n, h_dim), jnp.float32),
            pltpu.VMEM((N_DEV, m_per, h_dim), jnp.float32),
            pltpu.SemaphoreType.DMA((N_DEV - 1,)),
            pltpu.SemaphoreType.DMA((N_DEV - 1,)),
        ],
        compiler_params=pltpu.CompilerParams(collective_id=0),
    )(x, router_W, route_idx, expert_W)
